# baseline (speedup 1.0000x reference)
import functools

import jax
import jax.numpy as jnp
import numpy as np
from jax import lax
from jax.experimental import pallas as pl
from jax.experimental.pallas import tpu as pltpu
from jax.experimental.pallas import tpu_sc as plsc

_NC = 2
_NS = 16
_NW = _NC * _NS
_P = 16384
_PPW = _P // _NW
_CH = 16
_NCHUNK = _PPW // _CH
_NOUT = 451

_DNUMS = lax.GatherDimensionNumbers(
    offset_dims=(), collapsed_slice_dims=(0,), start_index_map=(0,))


def _axis_setup(c_ref, off, size):
  s = c_ref[pl.ds(off, _CH)] * np.float32(size)
  s = jnp.minimum(jnp.maximum(s, np.float32(0.01)), np.float32(size - 1.01))
  i1 = s.astype(jnp.int32)
  f1 = i1.astype(jnp.float32)
  frac = s - f1
  i2 = i1 + jnp.where(frac > np.float32(0.0), 1, 0).astype(jnp.int32)
  w_lo = i2.astype(jnp.float32) - s
  w_hi = frac
  return i1, w_lo, w_hi


def _wbcast(w, p):
  idx = jnp.full((16,), p, jnp.int32)
  return lax.gather(w, idx[:, None], dimension_numbers=_DNUMS,
                    slice_sizes=(1,),
                    mode=lax.GatherScatterMode.PROMISE_IN_BOUNDS)


def _pairs_setup(cx_v, cy_v, cz_v, off, size):
  x1, wx1, wx2 = _axis_setup(cx_v, off, size)
  y1, wy1, wy2 = _axis_setup(cy_v, off, size)
  z1, wz1, wz2 = _axis_setup(cz_v, off, size)
  x2i = x1 + 1
  y2i = y1 + 1
  pairs = ((x1, wx1, y1, wy1), (x1, wx1, y2i, wy2),
           (x2i, wx2, y1, wy1), (x2i, wx2, y2i, wy2))
  return z1, wz1, wz2, pairs


def _sc_body(t1, t2, t3, cx, cy, cz, mf, out,
             cx_v, cy_v, cz_v, mf_v, idx1, idx2, idx3,
             rows1, rows2, rows3, outv, sem1, sem2, sem3):
  wid = lax.axis_index("s") * _NC + lax.axis_index("c")
  base = wid * _PPW

  pltpu.sync_copy(cx.at[pl.ds(base, _PPW)], cx_v)
  pltpu.sync_copy(cy.at[pl.ds(base, _PPW)], cy_v)
  pltpu.sync_copy(cz.at[pl.ds(base, _PPW)], cz_v)
  pltpu.sync_copy(mf.at[pl.ds(3 * base, 3 * _PPW)],
                  mf_v.at[pl.ds(16, 3 * _PPW)])

  def fire1(i):
    off = i * _CH
    z1, _, _, pairs = _pairs_setup(cx_v, cy_v, cz_v, off, 32)
    q = z1 >> 1
    pz = z1 & np.int32(1)
    for kp, (xa, _, yb, _) in enumerate(pairs):
      strip = (xa * np.int32(32) + yb) * np.int32(16) + q
      idx1[pl.ds(kp * 32, _CH)] = strip
      idx1[pl.ds(kp * 32 + 16, _CH)] = strip + pz
    pltpu.async_copy(t1.at[idx1], rows1, sem1)

  def fire23(i, size, t_ref, idx_ref, rows_ref, sem):
    off = i * _CH
    z1, _, _, pairs = _pairs_setup(cx_v, cy_v, cz_v, off, size)
    s32 = np.int32(size)
    for kp, (xa, _, yb, _) in enumerate(pairs):
      vbase = (xa * s32 + yb) * s32 + z1
      idx_ref[pl.ds(kp * 32, _CH)] = vbase
      idx_ref[pl.ds(kp * 32 + 16, _CH)] = vbase + np.int32(1)
    pltpu.async_copy(t_ref.at[idx_ref], rows_ref, sem)

  def compute1(i):
    off = i * _CH
    z1, wz1, wz2, pairs = _pairs_setup(cx_v, cy_v, cz_v, off, 32)
    even = (z1 & np.int32(1)) == np.int32(0)
    zero = jnp.zeros((16,), jnp.float32)
    u0 = jnp.where(even, wz1, zero)
    u1 = jnp.where(even, wz2, wz1)
    u2 = jnp.where(even, zero, wz2)
    wxy = [wxa * wyb for (_, wxa, _, wyb) in pairs]

    @pl.loop(0, _CH)
    def _point(p):
      ub = [_wbcast(u, p) for u in (u0, u1, u2)]
      wb = [_wbcast(w, p) for w in wxy]
      w = [[wb[kp] * ub[s] for s in range(3)] for kp in range(4)]

      @pl.loop(0, 4, unroll=4)
      def _chanvec(j):
        lo = pl.ds(j * 16, 16)
        hi = pl.ds(64 + j * 16, 16)
        acc = (w[0][0] * rows1[p, lo] + w[0][1] * rows1[p, hi]
               + w[0][2] * rows1[_CH + p, lo])
        for kp in range(1, 4):
          r0 = kp * 32 + p
          acc = (acc + w[kp][0] * rows1[r0, lo] + w[kp][1] * rows1[r0, hi]
                 + w[kp][2] * rows1[r0 + _CH, lo])
        outv[p, lo] = acc
      outv[p, pl.ds(435, 16)] = mf_v[pl.ds(3 * (off + p) + 3, 16)]

  def compute23(i, size, col, nvec, unroll, rows_ref):
    off = i * _CH
    _, wz1, wz2, pairs = _pairs_setup(cx_v, cy_v, cz_v, off, size)
    wlo = [wxa * wyb * wz1 for (_, wxa, _, wyb) in pairs]
    whi = [wxa * wyb * wz2 for (_, wxa, _, wyb) in pairs]

    @pl.loop(0, _CH)
    def _point(p):
      ws = [(_wbcast(wlo[kp], p), _wbcast(whi[kp], p)) for kp in range(4)]

      @pl.loop(0, nvec, unroll=unroll)
      def _chanvec(j):
        sl = pl.ds(j * 16, 16)
        acc = ws[0][0] * rows_ref[p, sl] + ws[0][1] * rows_ref[_CH + p, sl]
        for kp in range(1, 4):
          r0 = kp * 32 + p
          acc = (acc + ws[kp][0] * rows_ref[r0, sl]
                 + ws[kp][1] * rows_ref[r0 + _CH, sl])
        outv[p, pl.ds(col + j * 16, 16)] = acc

  def wait(t_ref, rows_ref, sem):
    pltpu.make_async_copy(t_ref.at[pl.ds(0, rows_ref.shape[0])],
                          rows_ref, sem).wait()

  fire1(0)

  @pl.loop(0, _NCHUNK)
  def _sched(i):
    off = i * _CH
    fire23(i, 16, t2, idx2, rows2, sem2)
    wait(t1, rows1, sem1)
    compute1(i)
    fire23(i, 8, t3, idx3, rows3, sem3)
    wait(t2, rows2, sem2)
    compute23(i, 16, 64, 8, 4, rows2)

    @pl.when(i + 1 < _NCHUNK)
    def _():
      fire1(i + 1)

    wait(t3, rows3, sem3)
    compute23(i, 8, 192, 16, 4, rows3)
    pltpu.sync_copy(outv, out.at[pl.ds(base + off, _CH)])


@jax.jit
def _projection_sc(t1, t2, t3, cx, cy, cz, mf):
  mesh = plsc.VectorSubcoreMesh(core_axis_name="c", subcore_axis_name="s")
  out_type = jax.ShapeDtypeStruct((_P, _NOUT), jnp.float32)
  scratch = [
      pltpu.VMEM((_PPW,), jnp.float32),
      pltpu.VMEM((_PPW,), jnp.float32),
      pltpu.VMEM((_PPW,), jnp.float32),
      pltpu.VMEM((16 + 3 * _PPW,), jnp.float32),
      pltpu.VMEM((128,), jnp.int32),
      pltpu.VMEM((128,), jnp.int32),
      pltpu.VMEM((128,), jnp.int32),
      pltpu.VMEM((128, 128), jnp.float32),
      pltpu.VMEM((128, 128), jnp.float32),
      pltpu.VMEM((128, 256), jnp.float32),
      pltpu.VMEM((_CH, _NOUT), jnp.float32),
      pltpu.SemaphoreType.DMA,
      pltpu.SemaphoreType.DMA,
      pltpu.SemaphoreType.DMA,
  ]
  run = pl.kernel(_sc_body, out_type=out_type, mesh=mesh,
                  scratch_types=scratch,
                  compiler_params=pltpu.CompilerParams(
                      use_tc_tiling_on_sc=True))
  return run(t1, t2, t3, cx, cy, cz, mf)


def kernel(features0, features1, features2, features3, features4,
           mesh_coords, mesh_features):
  t1 = features1.reshape(16384, 128)
  t2 = features2.reshape(4096, 128)
  t3 = features3.reshape(512, 256)
  mc = mesh_coords[0]
  out = _projection_sc(t1, t2, t3, mc[:, 0], mc[:, 1], mc[:, 2],
                       mesh_features.reshape(3 * _P))
  return out[None]


# speedup vs baseline: 1.1882x; 1.1882x over previous
import functools

import jax
import jax.numpy as jnp
import numpy as np
from jax import lax
from jax.experimental import pallas as pl
from jax.experimental.pallas import tpu as pltpu
from jax.experimental.pallas import tpu_sc as plsc

_NC = 2
_NS = 16
_NW = _NC * _NS
_P = 16384
_PPW = _P // _NW
_CH = 16
_NCHUNK = _PPW // _CH
_NOUT = 451

_DNUMS = lax.GatherDimensionNumbers(
    offset_dims=(), collapsed_slice_dims=(0,), start_index_map=(0,))


def _axis_setup(c_ref, off, size):
  s = c_ref[pl.ds(off, _CH)] * np.float32(size)
  s = jnp.minimum(jnp.maximum(s, np.float32(0.01)), np.float32(size - 1.01))
  i1 = s.astype(jnp.int32)
  f1 = i1.astype(jnp.float32)
  frac = s - f1
  i2 = i1 + jnp.where(frac > np.float32(0.0), 1, 0).astype(jnp.int32)
  w_lo = i2.astype(jnp.float32) - s
  w_hi = frac
  return i1, w_lo, w_hi


def _wbcast(w, p):
  idx = jnp.full((16,), p, jnp.int32)
  return lax.gather(w, idx[:, None], dimension_numbers=_DNUMS,
                    slice_sizes=(1,),
                    mode=lax.GatherScatterMode.PROMISE_IN_BOUNDS)


def _bfpair(v):
  a = plsc.bitcast(jnp.left_shift(v, np.int32(16)), jnp.float32)
  b = plsc.bitcast(jnp.bitwise_and(v, np.int32(-65536)), jnp.float32)
  return a, b


def _pairs_setup(cx_v, cy_v, cz_v, off, size):
  x1, wx1, wx2 = _axis_setup(cx_v, off, size)
  y1, wy1, wy2 = _axis_setup(cy_v, off, size)
  z1, wz1, wz2 = _axis_setup(cz_v, off, size)
  x2i = x1 + 1
  y2i = y1 + 1
  pairs = ((x1, wx1, y1, wy1), (x1, wx1, y2i, wy2),
           (x2i, wx2, y1, wy1), (x2i, wx2, y2i, wy2))
  return z1, wz1, wz2, pairs


def _sc_body(t1, t2, t3, cx, cy, cz, mf, out,
             cx_v, cy_v, cz_v, mf_v, idx1, idx2, idx3,
             rows1, rows2, rows3, outv, sem1, sem2, sem3):
  wid = lax.axis_index("s") * _NC + lax.axis_index("c")
  base = wid * _PPW

  pltpu.sync_copy(cx.at[pl.ds(base, _PPW)], cx_v)
  pltpu.sync_copy(cy.at[pl.ds(base, _PPW)], cy_v)
  pltpu.sync_copy(cz.at[pl.ds(base, _PPW)], cz_v)
  pltpu.sync_copy(mf.at[pl.ds(3 * base, 3 * _PPW)],
                  mf_v.at[pl.ds(16, 3 * _PPW)])

  def fire1(i):
    off = i * _CH
    z1, _, _, pairs = _pairs_setup(cx_v, cy_v, cz_v, off, 32)
    q = z1 >> 1
    pz = z1 & np.int32(1)
    for kp, (xa, _, yb, _) in enumerate(pairs):
      strip = (xa * np.int32(32) + yb) * np.int32(16) + q
      idx1[pl.ds(kp * 32, _CH)] = strip
      idx1[pl.ds(kp * 32 + 16, _CH)] = strip + pz
    pltpu.async_copy(t1.at[idx1], rows1, sem1)

  def fire2(i):
    off = i * _CH
    z1, _, _, pairs = _pairs_setup(cx_v, cy_v, cz_v, off, 16)
    for kp, (xa, _, yb, _) in enumerate(pairs):
      idx2[pl.ds(kp * _CH, _CH)] = (xa * np.int32(16) + yb) * np.int32(15) + z1
    pltpu.async_copy(t2.at[idx2], rows2, sem2)

  def fire3(i):
    off = i * _CH
    z1, _, _, pairs = _pairs_setup(cx_v, cy_v, cz_v, off, 8)
    for kp, (xa, _, yb, _) in enumerate(pairs):
      vbase = (xa * np.int32(8) + yb) * np.int32(8) + z1
      idx3[pl.ds(kp * 32, _CH)] = vbase
      idx3[pl.ds(kp * 32 + 16, _CH)] = vbase + np.int32(1)
    pltpu.async_copy(t3.at[idx3], rows3, sem3)

  def compute1(i):
    off = i * _CH
    z1, wz1, wz2, pairs = _pairs_setup(cx_v, cy_v, cz_v, off, 32)
    even = (z1 & np.int32(1)) == np.int32(0)
    zero = jnp.zeros((16,), jnp.float32)
    u0 = jnp.where(even, wz1, zero)
    u1 = jnp.where(even, wz2, wz1)
    u2 = jnp.where(even, zero, wz2)
    wxy = [wxa * wyb for (_, wxa, _, wyb) in pairs]

    @pl.loop(0, _CH)
    def _point(p):
      ub = [_wbcast(u, p) for u in (u0, u1, u2)]
      wb = [_wbcast(w, p) for w in wxy]
      w = [[wb[kp] * ub[s] for s in range(3)] for kp in range(4)]

      for j in range(4):
        lo = pl.ds(j * 16, 16)
        hi = pl.ds(64 + j * 16, 16)
        acc = None
        for kp in range(4):
          r0 = kp * 32 + p
          t = (w[kp][0] * rows1[r0, lo] + w[kp][1] * rows1[r0, hi]
               + w[kp][2] * rows1[r0 + _CH, lo])
          acc = t if acc is None else acc + t
        outv[p, lo] = acc
      outv[p, pl.ds(435, 16)] = mf_v[pl.ds(3 * (off + p) + 3, 16)]

  def compute2(i):
    off = i * _CH
    _, wz1, wz2, pairs = _pairs_setup(cx_v, cy_v, cz_v, off, 16)
    wlo = [wxa * wyb * wz1 for (_, wxa, _, wyb) in pairs]
    whi = [wxa * wyb * wz2 for (_, wxa, _, wyb) in pairs]

    @pl.loop(0, _CH)
    def _point(p):
      ws = [(_wbcast(wlo[kp], p), _wbcast(whi[kp], p)) for kp in range(4)]
      for g in range(4):
        sla = pl.ds(g * 16, 16)
        slb = pl.ds(64 + g * 16, 16)
        acc_a = None
        acc_b = None
        for kp in range(4):
          r0 = kp * _CH + p
          a1, b1 = _bfpair(rows2[r0, sla])
          a2, b2 = _bfpair(rows2[r0, slb])
          pa = ws[kp][0] * a1 + ws[kp][1] * a2
          pb = ws[kp][0] * b1 + ws[kp][1] * b2
          acc_a = pa if acc_a is None else acc_a + pa
          acc_b = pb if acc_b is None else acc_b + pb
        outv[p, pl.ds(64 + g * 32, 16)] = acc_a
        outv[p, pl.ds(64 + g * 32 + 16, 16)] = acc_b

  def compute3(i):
    off = i * _CH
    _, wz1, wz2, pairs = _pairs_setup(cx_v, cy_v, cz_v, off, 8)
    wlo = [wxa * wyb * wz1 for (_, wxa, _, wyb) in pairs]
    whi = [wxa * wyb * wz2 for (_, wxa, _, wyb) in pairs]

    @pl.loop(0, _CH)
    def _point(p):
      ws = [(_wbcast(wlo[kp], p), _wbcast(whi[kp], p)) for kp in range(4)]
      for g in range(8):
        sl = pl.ds(g * 16, 16)
        acc_a = None
        acc_b = None
        for kp in range(4):
          r0 = kp * 32 + p
          a1, b1 = _bfpair(rows3[r0, sl])
          a2, b2 = _bfpair(rows3[r0 + _CH, sl])
          pa = ws[kp][0] * a1 + ws[kp][1] * a2
          pb = ws[kp][0] * b1 + ws[kp][1] * b2
          acc_a = pa if acc_a is None else acc_a + pa
          acc_b = pb if acc_b is None else acc_b + pb
        outv[p, pl.ds(192 + g * 32, 16)] = acc_a
        outv[p, pl.ds(192 + g * 32 + 16, 16)] = acc_b

  def wait(t_ref, idx_ref, rows_ref, sem):
    pltpu.make_async_copy(t_ref.at[idx_ref], rows_ref, sem).wait()

  fire1(0)

  @pl.loop(0, _NCHUNK)
  def _sched(i):
    off = i * _CH
    fire2(i)
    wait(t1, idx1, rows1, sem1)
    compute1(i)
    fire3(i)
    wait(t2, idx2, rows2, sem2)
    compute2(i)

    @pl.when(i + 1 < _NCHUNK)
    def _():
      fire1(i + 1)

    wait(t3, idx3, rows3, sem3)
    compute3(i)
    pltpu.sync_copy(outv, out.at[pl.ds(base + off, _CH)])


@jax.jit
def _projection_sc(t1, t2, t3, cx, cy, cz, mf):
  mesh = plsc.VectorSubcoreMesh(core_axis_name="c", subcore_axis_name="s")
  out_type = jax.ShapeDtypeStruct((_P, _NOUT), jnp.float32)
  scratch = [
      pltpu.VMEM((_PPW,), jnp.float32),
      pltpu.VMEM((_PPW,), jnp.float32),
      pltpu.VMEM((_PPW,), jnp.float32),
      pltpu.VMEM((16 + 3 * _PPW,), jnp.float32),
      pltpu.VMEM((128,), jnp.int32),
      pltpu.VMEM((64,), jnp.int32),
      pltpu.VMEM((128,), jnp.int32),
      pltpu.VMEM((128, 128), jnp.float32),
      pltpu.VMEM((64, 128), jnp.int32),
      pltpu.VMEM((128, 128), jnp.int32),
      pltpu.VMEM((_CH, _NOUT), jnp.float32),
      pltpu.SemaphoreType.DMA,
      pltpu.SemaphoreType.DMA,
      pltpu.SemaphoreType.DMA,
  ]
  run = pl.kernel(_sc_body, out_type=out_type, mesh=mesh,
                  scratch_types=scratch,
                  compiler_params=pltpu.CompilerParams(
                      needs_layout_passes=False))
  return run(t1, t2, t3, cx, cy, cz, mf)


def _perm(c):
  s = np.arange(c)
  g = s // 32
  r = s % 32
  return g * 32 + np.where(r % 2 == 0, r // 2, 16 + r // 2)


def _pack_i32(x_bf16_2d):
  n, c = x_bf16_2d.shape
  xp = x_bf16_2d[:, _perm(c)]
  return lax.bitcast_convert_type(xp.reshape(n, c // 2, 2), jnp.int32)


def kernel(features0, features1, features2, features3, features4,
           mesh_coords, mesh_features):
  t1 = features1.reshape(16384, 128)
  f2 = features2[0].astype(jnp.bfloat16)
  f2z = jnp.concatenate([f2[:, :, :-1, :], f2[:, :, 1:, :]], axis=-1)
  t2 = _pack_i32(f2z.reshape(16 * 16 * 15, 256))
  t3 = _pack_i32(features3.astype(jnp.bfloat16).reshape(512, 256))
  mc = mesh_coords[0]
  out = _projection_sc(t1, t2, t3, mc[:, 0], mc[:, 1], mc[:, 2],
                       mesh_features.reshape(3 * _P))
  return out[None]
